# Initial kernel scaffold; baseline (speedup 1.0000x reference)
import jax
import jax.numpy as jnp
from jax.experimental import pallas as pl


def kernel(x, edge_attr, edge_nei, kge_x, params, edge_index, batch, kge_batch):
    raise NotImplementedError("write your pallas kernel here")


# speedup vs baseline: 1.2922x; 1.2922x over previous
import functools
import math

import jax
import jax.numpy as jnp
from jax import lax
from jax.experimental import pallas as pl
from jax.experimental.pallas import tpu as pltpu
from jax.experimental.pallas import tpu_sc as plsc

NF = 256
BINS = 256
NUM_GRAPHS = 512
N_NODES = 10000
N_EDGES = 160000

ET = 640
NT = 400
EG = N_EDGES // ET
NG = N_NODES // NT
INV_SQRT_NF = 1.0 / math.sqrt(NF)

F32 = jnp.float32


def _dot(a, b):
    return jax.lax.dot_general(a, b, (((1,), (0,)), ((), ())),
                               precision=jax.lax.Precision.HIGHEST,
                               preferred_element_type=F32)


def _blk(shape):
    return pl.BlockSpec(shape, lambda t: (0,) * len(shape))


def _row_blk(shape):
    return pl.BlockSpec(shape, lambda t: (t,) + (0,) * (len(shape) - 1))


def _embed_body(x_ref, w_ref, b_ref, out_ref):
    v = _dot(x_ref[...], w_ref[...]) + b_ref[...]
    out_ref[0] = v[:, :128]
    out_ref[1] = v[:, 128:]


def _tc_embed(xp, w, b):
    return pl.pallas_call(
        _embed_body,
        grid=(NG,),
        in_specs=[_row_blk((NT, 128)), _blk((128, NF)), _blk((1, NF))],
        out_specs=pl.BlockSpec((2, NT, 128), lambda t: (0, t, 0)),
        out_shape=jax.ShapeDtypeStruct((2, N_NODES, 128), F32),
    )(xp, w, b)


def _rbf_row(d, centers, gamma, w_ref, b_ref):
    rb = jnp.exp(-gamma * (d - centers) ** 2)
    return jax.nn.softplus(
        _dot(rb, w_ref[...]) + b_ref[...])


def _rbf_ef_body(ea_ref, c_ref, w_ref, b_ref, out_ref):
    ea = ea_ref[...]
    d = -0.75 / jnp.sqrt(ea[:, 0:1] ** 2 + ea[:, 1:2] ** 2 + ea[:, 2:3] ** 2)
    gamma = (BINS - 1) / 4.0
    out_ref[...] = _rbf_row(d, c_ref[...], gamma, w_ref, b_ref)


def _tc_rbf_ef(edge_attr, centers, w, b):
    return pl.pallas_call(
        _rbf_ef_body,
        grid=(EG,),
        in_specs=[_row_blk((ET, 3)), _blk((1, BINS)), _blk((BINS, NF)),
                  _blk((1, NF))],
        out_specs=_row_blk((ET, NF)),
        out_shape=jax.ShapeDtypeStruct((N_EDGES, NF), F32),
    )(edge_attr, centers, w, b)


def _rbf_nlna_body(nei_ref, ea_ref, cl_ref, ca_ref, wl_ref, bl_ref, wa_ref,
                   ba_ref, nl_ref, na_ref):
    ea = ea_ref[...]
    ea_norm = jnp.sqrt(ea[:, 0:1] ** 2 + ea[:, 1:2] ** 2 + ea[:, 2:3] ** 2)
    nei = nei_ref[...]
    gam_l = (BINS - 1) / 4.0
    gam_a = (BINS - 1) / 2.0
    for i in range(3):
        nx = nei[:, 3 * i:3 * i + 1]
        ny = nei[:, 3 * i + 1:3 * i + 2]
        nz = nei[:, 3 * i + 2:3 * i + 3]
        nn = jnp.sqrt(nx * nx + ny * ny + nz * nz)
        dlen = -0.75 / nn
        dotp = nx * ea[:, 0:1] + ny * ea[:, 1:2] + nz * ea[:, 2:3]
        cos = jnp.clip(dotp / (nn * ea_norm), -1.0, 1.0)
        nl_ref[:, i * NF:(i + 1) * NF] = _rbf_row(dlen, cl_ref[...], gam_l,
                                                  wl_ref, bl_ref)
        na_ref[:, i * NF:(i + 1) * NF] = _rbf_row(cos, ca_ref[...], gam_a,
                                                  wa_ref, ba_ref)


def _tc_rbf_nlna(nei9, edge_attr, c_len, c_ang, wl, bl, wa, ba):
    return pl.pallas_call(
        _rbf_nlna_body,
        grid=(EG,),
        in_specs=[_row_blk((ET, 9)), _row_blk((ET, 3)), _blk((1, BINS)),
                  _blk((1, BINS)), _blk((BINS, NF)), _blk((1, NF)),
                  _blk((BINS, NF)), _blk((1, NF))],
        out_specs=[_row_blk((ET, 3 * NF)), _row_blk((ET, 3 * NF))],
        out_shape=[jax.ShapeDtypeStruct((N_EDGES, 3 * NF), F32),
                   jax.ShapeDtypeStruct((N_EDGES, 3 * NF), F32)],
    )(nei9, edge_attr, c_len, c_ang, wl, bl, wa, ba)


def _pack2(a, b):
    ua = jax.lax.bitcast_convert_type(a, jnp.uint32) + jnp.uint32(0x8000)
    ub = jax.lax.bitcast_convert_type(b, jnp.uint32) + jnp.uint32(0x8000)
    return (ua >> 16) | (ub & jnp.uint32(0xFFFF0000))


def _unpack2(w):
    a = jax.lax.bitcast_convert_type(w << 16, F32)
    b = jax.lax.bitcast_convert_type(w & jnp.uint32(0xFFFF0000), F32)
    return a, b


def _prep_body(n_ref, wd_ref, bd_ref, ws_ref, bs_ref, td_ref, ts_ref):
    node = jnp.concatenate([n_ref[0], n_ref[1]], axis=1)
    td = _dot(node, wd_ref[...]) + bd_ref[...]
    ts = _dot(node, ws_ref[...]) + bs_ref[...]
    td_ref[...] = _pack2(td[:, :384], td[:, 384:])
    ts_ref[...] = _pack2(ts[:, :256], ts[:, 256:])


def _tc_prep(node_st, wd, bd, ws, bs):
    return pl.pallas_call(
        _prep_body,
        grid=(NG,),
        in_specs=[pl.BlockSpec((2, NT, 128), lambda t: (0, t, 0)),
                  _blk((NF, 3 * NF)), _blk((1, 3 * NF)),
                  _blk((NF, 2 * NF)), _blk((1, 2 * NF))],
        out_specs=[_row_blk((NT, 384)), _row_blk((NT, 256))],
        out_shape=[jax.ShapeDtypeStruct((N_NODES, 384), jnp.uint32),
                   jax.ShapeDtypeStruct((N_NODES, 256), jnp.uint32)],
    )(node_st, wd, bd, ws, bs)


def _conv_p1_body(gd_ref, gs_ref, ef_ref, ek_ref, em_ref, k2_ref,
                  m2_ref, bias_ref, alpha_ref, msg_ref, st_ref):
    t = pl.program_id(0)
    da, db = _unpack2(gd_ref[...])
    sa, sb = _unpack2(gs_ref[...])
    ef = ef_ref[...]
    bias = bias_ref[...]
    qd = da[:, :NF]
    kd = jnp.concatenate([da[:, NF:], db[:, :128]], axis=1)
    vd = db[:, 128:]
    hk = jax.nn.silu(kd + sa + _dot(ef, ek_ref[...]) + bias[0:1, :])
    alpha = qd * (_dot(hk, k2_ref[...]) + bias[2:3, :]) * INV_SQRT_NF
    hm = jax.nn.silu(vd + sb + _dot(ef, em_ref[...]) + bias[1:2, :])
    msg = _dot(hm, m2_ref[...]) + bias[3:4, :]
    alpha_ref[...] = alpha
    msg_ref[...] = msg

    @pl.when(t == 0)
    def _():
        st_ref[...] = jnp.zeros_like(st_ref)
        st_ref[2:3, :] = jnp.mean(alpha, axis=0, keepdims=True)

    c0 = st_ref[2:3, :]
    d = alpha - c0
    st_ref[0:1, :] += jnp.sum(d, axis=0, keepdims=True)
    st_ref[1:2, :] += jnp.sum(d * d, axis=0, keepdims=True)


def _tc_conv_p1(gd, gs, ef, ek, em, k2, m2, bias):
    return pl.pallas_call(
        _conv_p1_body,
        grid=(EG,),
        in_specs=[_row_blk((ET, 384)), _row_blk((ET, 256)),
                  _row_blk((ET, NF)), _blk((NF, NF)), _blk((NF, NF)),
                  _blk((NF, NF)), _blk((NF, NF)), _blk((8, NF))],
        out_specs=[_row_blk((ET, NF)), _row_blk((ET, NF)), _blk((8, NF))],
        out_shape=[jax.ShapeDtypeStruct((N_EDGES, NF), F32),
                   jax.ShapeDtypeStruct((N_EDGES, NF), F32),
                   jax.ShapeDtypeStruct((8, NF), F32)],
    )(gd, gs, ef, ek, em, k2, m2, bias)


def _conv_p2_body(alpha_ref, msg_ref, ss_ref, out_ref):
    ss = ss_ref[...]
    gate = jax.nn.sigmoid(alpha_ref[...] * ss[0:1, :] + ss[1:2, :])
    gated = msg_ref[...] * gate
    out_ref[0] = gated[:, :128]
    out_ref[1] = gated[:, 128:]


def _tc_conv_p2(alpha, msg, ss):
    return pl.pallas_call(
        _conv_p2_body,
        grid=(EG,),
        in_specs=[_row_blk((ET, NF)), _row_blk((ET, NF)), _blk((8, NF))],
        out_specs=pl.BlockSpec((2, ET, 128), lambda t: (0, t, 0)),
        out_shape=jax.ShapeDtypeStruct((2, N_EDGES, 128), F32),
    )(alpha, msg, ss)


def _fina_body(agg_ref, wc_ref, bc_ref, z_ref, st_ref):
    t = pl.program_id(0)
    agg = jnp.concatenate([agg_ref[0], agg_ref[1]], axis=1)
    z = _dot(agg, wc_ref[...]) + bc_ref[...]
    z_ref[...] = z

    @pl.when(t == 0)
    def _():
        st_ref[...] = jnp.zeros_like(st_ref)
        st_ref[2:3, :] = jnp.mean(z, axis=0, keepdims=True)

    c0 = st_ref[2:3, :]
    d = z - c0
    st_ref[0:1, :] += jnp.sum(d, axis=0, keepdims=True)
    st_ref[1:2, :] += jnp.sum(d * d, axis=0, keepdims=True)


def _tc_fina(agg_st, wc, bc):
    return pl.pallas_call(
        _fina_body,
        grid=(NG,),
        in_specs=[pl.BlockSpec((2, NT, 128), lambda t: (0, t, 0)),
                  _blk((NF, NF)), _blk((1, NF))],
        out_specs=[_row_blk((NT, NF)), _blk((8, NF))],
        out_shape=[jax.ShapeDtypeStruct((N_NODES, NF), F32),
                   jax.ShapeDtypeStruct((8, NF), F32)],
    )(agg_st, wc, bc)


def _finb_body(n_ref, z_ref, ss_ref, out_ref):
    node = jnp.concatenate([n_ref[0], n_ref[1]], axis=1)
    ss = ss_ref[...]
    res = jax.nn.softplus(node + z_ref[...] * ss[0:1, :] + ss[1:2, :])
    out_ref[0] = res[:, :128]
    out_ref[1] = res[:, 128:]


def _tc_finb(node_st, z, ss):
    return pl.pallas_call(
        _finb_body,
        grid=(NG,),
        in_specs=[pl.BlockSpec((2, NT, 128), lambda t: (0, t, 0)),
                  _row_blk((NT, NF)), _blk((8, NF))],
        out_specs=pl.BlockSpec((2, NT, 128), lambda t: (0, t, 0)),
        out_shape=jax.ShapeDtypeStruct((2, N_NODES, 128), F32),
    )(node_st, z, ss)


def _eu_slot1_body(ef_ref, nl_ref, na_ref, wcat_ref, wq_ref, k2_ref,
                   bias_ref, alpha_ref, st_ref):
    t = pl.program_id(0)
    ef = ef_ref[...]
    bias = bias_ref[...]
    qx = _dot(ef, wq_ref[...]) + bias[6:7, :]
    pre = (_dot(ef, wcat_ref[0:NF]) + _dot(nl_ref[...], wcat_ref[NF:2 * NF])
           + _dot(na_ref[...], wcat_ref[2 * NF:]) + bias[0:1, :])
    hk = jax.nn.silu(pre)
    a = qx * (_dot(hk, k2_ref[...]) + bias[7:8, :]) * INV_SQRT_NF
    alpha_ref[...] = a

    @pl.when(t == 0)
    def _():
        st_ref[...] = jnp.zeros_like(st_ref)
        st_ref[2:3, :] = jnp.mean(a, axis=0, keepdims=True)

    c0 = st_ref[2:3, :]
    d = a - c0
    st_ref[0:1, :] += jnp.sum(d, axis=0, keepdims=True)
    st_ref[1:2, :] += jnp.sum(d * d, axis=0, keepdims=True)


def _tc_eu_slot1(ef, nl, na, i, wcat, wq, k2, bias):
    return pl.pallas_call(
        _eu_slot1_body,
        grid=(EG,),
        in_specs=[_row_blk((ET, NF)),
                  pl.BlockSpec((ET, NF), lambda t, i=i: (t, i)),
                  pl.BlockSpec((ET, NF), lambda t, i=i: (t, i)),
                  _blk((3 * NF, NF)), _blk((NF, NF)), _blk((NF, NF)),
                  _blk((8, NF))],
        out_specs=[_row_blk((ET, NF)), _blk((8, NF))],
        out_shape=[jax.ShapeDtypeStruct((N_EDGES, NF), F32),
                   jax.ShapeDtypeStruct((8, NF), F32)],
    )(ef, nl, na, wcat, wq, k2, bias)


def _eu_slot2_body(ef_ref, nl_ref, na_ref, al_ref, sp_ref, ss_ref, wcat_ref,
                   m2_ref, bias_ref, s_ref):
    ef = ef_ref[...]
    bias = bias_ref[...]
    ss = ss_ref[...]
    pre = (_dot(ef, wcat_ref[0:NF]) + _dot(nl_ref[...], wcat_ref[NF:2 * NF])
           + _dot(na_ref[...], wcat_ref[2 * NF:]) + bias[0:1, :])
    hm = jax.nn.silu(pre)
    m = _dot(hm, m2_ref[...]) + bias[6:7, :]
    gate = jax.nn.sigmoid(al_ref[...] * ss[0:1, :] + ss[1:2, :])
    s_ref[...] = sp_ref[...] + m * gate


def _eu_slot2_first_body(ef_ref, nl_ref, na_ref, al_ref, ss_ref, wcat_ref,
                         m2_ref, bias_ref, s_ref):
    ef = ef_ref[...]
    bias = bias_ref[...]
    ss = ss_ref[...]
    pre = (_dot(ef, wcat_ref[0:NF]) + _dot(nl_ref[...], wcat_ref[NF:2 * NF])
           + _dot(na_ref[...], wcat_ref[2 * NF:]) + bias[0:1, :])
    hm = jax.nn.silu(pre)
    m = _dot(hm, m2_ref[...]) + bias[6:7, :]
    gate = jax.nn.sigmoid(al_ref[...] * ss[0:1, :] + ss[1:2, :])
    s_ref[...] = m * gate


def _tc_eu_slot2(ef, nl, na, alpha_i, s_prev, i, ss, wcat, m2, bias):
    slot_specs = [_row_blk((ET, NF)),
                  pl.BlockSpec((ET, NF), lambda t, i=i: (t, i)),
                  pl.BlockSpec((ET, NF), lambda t, i=i: (t, i)),
                  _row_blk((ET, NF))]
    tail_specs = [_blk((8, NF)), _blk((3 * NF, NF)), _blk((NF, NF)),
                  _blk((8, NF))]
    if s_prev is None:
        return pl.pallas_call(
            _eu_slot2_first_body,
            grid=(EG,),
            in_specs=slot_specs + tail_specs,
            out_specs=_row_blk((ET, NF)),
            out_shape=jax.ShapeDtypeStruct((N_EDGES, NF), F32),
        )(ef, nl, na, alpha_i, ss, wcat, m2, bias)
    return pl.pallas_call(
        _eu_slot2_body,
        grid=(EG,),
        in_specs=slot_specs + [_row_blk((ET, NF))] + tail_specs,
        out_specs=_row_blk((ET, NF)),
        out_shape=jax.ShapeDtypeStruct((N_EDGES, NF), F32),
    )(ef, nl, na, alpha_i, s_prev, ss, wcat, m2, bias)


def _eu_fin_body(s_ref, wc_ref, bias_ref, out_ref, st_ref):
    t = pl.program_id(0)
    out = _dot(s_ref[...], wc_ref[...]) + bias_ref[7:8, :]
    out_ref[...] = out

    @pl.when(t == 0)
    def _():
        st_ref[...] = jnp.zeros_like(st_ref)
        st_ref[2:3, :] = jnp.mean(out, axis=0, keepdims=True)

    c0 = st_ref[2:3, :]
    d = out - c0
    st_ref[0:1, :] += jnp.sum(d, axis=0, keepdims=True)
    st_ref[1:2, :] += jnp.sum(d * d, axis=0, keepdims=True)


def _tc_eu_fin(s, wc, bias):
    return pl.pallas_call(
        _eu_fin_body,
        grid=(EG,),
        in_specs=[_row_blk((ET, NF)), _blk((NF, NF)), _blk((8, NF))],
        out_specs=[_row_blk((ET, NF)), _blk((8, NF))],
        out_shape=[jax.ShapeDtypeStruct((N_EDGES, NF), F32),
                   jax.ShapeDtypeStruct((8, NF), F32)],
    )(s, wc, bias)


def _eu_p3_body(ef_ref, op_ref, ss_ref, out_ref):
    ss = ss_ref[...]
    out_ref[...] = jax.nn.softplus(
        ef_ref[...] + op_ref[...] * ss[0:1, :] + ss[1:2, :])


def _tc_eu_p3(ef, out_pre, ss):
    return pl.pallas_call(
        _eu_p3_body,
        grid=(EG,),
        in_specs=[_row_blk((ET, NF)), _row_blk((ET, NF)), _blk((8, NF))],
        out_specs=_row_blk((ET, NF)),
        out_shape=jax.ShapeDtypeStruct((N_EDGES, NF), F32),
    )(ef, out_pre, ss)


def _proj_body(x_ref, w_ref, b_ref, out_ref):
    v = jax.nn.softplus(
        _dot(x_ref[...], w_ref[...])
        + b_ref[...])
    out_ref[0] = v[:, :128]
    out_ref[1] = v[:, 128:]


def _tc_proj(kge_x, w, b):
    return pl.pallas_call(
        _proj_body,
        grid=(NG,),
        in_specs=[_row_blk((NT, 128)), _blk((128, NF)), _blk((1, NF))],
        out_specs=pl.BlockSpec((2, NT, 128), lambda t: (0, t, 0)),
        out_shape=jax.ShapeDtypeStruct((2, N_NODES, 128), F32),
    )(kge_x, w, b)


def _head_body(gs_ref, gc_ref, es_ref, ec_ref, ca1w_ref, ca1b_ref, ca2w_ref,
               hv_ref, convw_ref, fcw_ref, out_ref):
    gcnt = jnp.maximum(gc_ref[0, :, 0:1], 1.0)
    gf = jnp.concatenate([gs_ref[0], gs_ref[1]], axis=1) / gcnt
    ecnt = jnp.maximum(ec_ref[0, :, 0:1], 1.0)
    el = jnp.concatenate([es_ref[0], es_ref[1]], axis=1) / ecnt
    hv = hv_ref[...]
    initial = gf + el
    h = jax.nn.relu(
        _dot(initial, ca1w_ref[...])
        + ca1b_ref[...])
    cattn = _dot(h, ca2w_ref[...]) + hv[3:4, :]
    mean = jnp.mean(initial, axis=-1, keepdims=True)
    mx = jnp.max(initial, axis=-1, keepdims=True)
    sattn = mean * hv[4:5, :] + mx * hv[5:6, :] + hv[6:7, :]
    pattn2 = jax.nn.sigmoid(initial * hv[0:1, :]
                            + (sattn + cattn) * hv[1:2, :] + hv[2:3, :])
    result = initial + pattn2 * gf + (1.0 - pattn2) * el
    feats = _dot(result, convw_ref[...]) + hv[7:8, :]
    feats = jax.nn.silu(
        _dot(feats, fcw_ref[...]) + hv[8:9, :])
    out_ref[...] = jnp.sum(feats * hv[9:10, :], axis=1, keepdims=True) + hv[10:11, :1]


def _tc_head(gsum, gcnt, esum, ecnt, ca1w, ca1b, ca2w, hv, convw, fcw):
    return pl.pallas_call(
        _head_body,
        grid=(1,),
        in_specs=[pl.BlockSpec((2, NUM_GRAPHS, 128), lambda t: (0, 0, 0)),
                  pl.BlockSpec((2, NUM_GRAPHS, 128), lambda t: (0, 0, 0)),
                  pl.BlockSpec((2, NUM_GRAPHS, 128), lambda t: (0, 0, 0)),
                  pl.BlockSpec((2, NUM_GRAPHS, 128), lambda t: (0, 0, 0)),
                  _blk((NF, 32)), _blk((1, 32)), _blk((32, NF)),
                  _blk((16, NF)), _blk((NF, NF)), _blk((NF, NF))],
        out_specs=_blk((NUM_GRAPHS, 1)),
        out_shape=jax.ShapeDtypeStruct((NUM_GRAPHS, 1), F32),
    )(gsum, gcnt, esum, ecnt, ca1w, ca1b, ca2w, hv, convw, fcw)


def _sc_mesh():
    return plsc.VectorSubcoreMesh(core_axis_name="c", subcore_axis_name="s")


_NW = 32
_CH = 40
_EPW = N_EDGES // _NW
_GCH = _EPW // _CH
_EPS = N_EDGES // 16
_SCH = _EPS // _CH
_NCH = N_NODES // _CH
_NPAD = 10240
_NPS = _NPAD // 16


def _sc_gather(tdp, tsp, dst, src):
    @functools.partial(
        pl.kernel, mesh=_sc_mesh(),
        out_type=[jax.ShapeDtypeStruct((N_EDGES, 384), jnp.uint32),
                  jax.ShapeDtypeStruct((N_EDGES, 256), jnp.uint32)],
        scratch_types=[pltpu.VMEM((_EPW,), jnp.int32),
                       pltpu.VMEM((_EPW,), jnp.int32),
                       pltpu.VMEM((_CH, 384), jnp.uint32),
                       pltpu.VMEM((_CH, 384), jnp.uint32),
                       pltpu.VMEM((_CH, 256), jnp.uint32),
                       pltpu.VMEM((_CH, 256), jnp.uint32),
                       pltpu.SemaphoreType.DMA,
                       pltpu.SemaphoreType.DMA,
                       pltpu.SemaphoreType.DMA,
                       pltpu.SemaphoreType.DMA],
    )
    def k(td_hbm, ts_hbm, dst_hbm, src_hbm, gd_hbm, gs_hbm,
          idxd, idxs, rd0, rd1, rs0, rs1, semd0, semd1, sems0, sems1):
        wid = lax.axis_index("c") * 16 + lax.axis_index("s")
        base = wid * _EPW
        pltpu.sync_copy(dst_hbm.at[pl.ds(base, _EPW)], idxd)
        pltpu.sync_copy(src_hbm.at[pl.ds(base, _EPW)], idxs)
        pltpu.async_copy(td_hbm.at[idxd.at[pl.ds(0, _CH)]], rd0, semd0)
        pltpu.async_copy(ts_hbm.at[idxs.at[pl.ds(0, _CH)]], rs0, sems0)

        def body(j, carry):
            @pl.when(j + 1 < _GCH)
            def _():
                o = (j + 1) * _CH

                @pl.when((j + 1) % 2 == 0)
                def _():
                    pltpu.async_copy(td_hbm.at[idxd.at[pl.ds(o, _CH)]], rd0,
                                     semd0)
                    pltpu.async_copy(ts_hbm.at[idxs.at[pl.ds(o, _CH)]], rs0,
                                     sems0)

                @pl.when((j + 1) % 2 == 1)
                def _():
                    pltpu.async_copy(td_hbm.at[idxd.at[pl.ds(o, _CH)]], rd1,
                                     semd1)
                    pltpu.async_copy(ts_hbm.at[idxs.at[pl.ds(o, _CH)]], rs1,
                                     sems1)

            o = j * _CH
            r0 = base + o

            @pl.when(j % 2 == 0)
            def _():
                pltpu.make_async_copy(td_hbm.at[idxd.at[pl.ds(o, _CH)]], rd0,
                                      semd0).wait()
                pltpu.sync_copy(rd0, gd_hbm.at[pl.ds(r0, _CH)])
                pltpu.make_async_copy(ts_hbm.at[idxs.at[pl.ds(o, _CH)]], rs0,
                                      sems0).wait()
                pltpu.sync_copy(rs0, gs_hbm.at[pl.ds(r0, _CH)])

            @pl.when(j % 2 == 1)
            def _():
                pltpu.make_async_copy(td_hbm.at[idxd.at[pl.ds(o, _CH)]], rd1,
                                      semd1).wait()
                pltpu.sync_copy(rd1, gd_hbm.at[pl.ds(r0, _CH)])
                pltpu.make_async_copy(ts_hbm.at[idxs.at[pl.ds(o, _CH)]], rs1,
                                      sems1).wait()
                pltpu.sync_copy(rs1, gs_hbm.at[pl.ds(r0, _CH)])

            return carry

        lax.fori_loop(0, _GCH, body, 0)

    return k(tdp, tsp, dst, src)


def _sc_scatter(gated_st, dst, zeros_np):
    @functools.partial(
        pl.kernel, mesh=_sc_mesh(),
        out_type=jax.ShapeDtypeStruct((2, _NPAD, 128), F32),
        scratch_types=[pltpu.VMEM((_SCH, _CH), jnp.int32),
                       pltpu.VMEM((_CH, 128), F32),
                       pltpu.VMEM((_CH, 128), F32),
                       pltpu.SemaphoreType.DMA,
                       pltpu.SemaphoreType.DMA,
                       pltpu.VMEM_SHARED((_NPAD, 128), F32)],
    )
    def k(g_hbm, dst_hbm, z_hbm, agg_hbm, idx, r0buf, r1buf, sem0, sem1, acc):
        c = lax.axis_index("c")
        s = lax.axis_index("s")
        base = s * _EPS
        pltpu.sync_copy(z_hbm, acc.at[pl.ds(s * _NPS, _NPS)])
        pltpu.sync_copy(dst_hbm.at[s], idx)
        plsc.subcore_barrier()
        pltpu.async_copy(g_hbm.at[c, pl.ds(base, _CH)], r0buf, sem0)

        def body(j, carry):
            @pl.when(j + 1 < _SCH)
            def _():
                o = base + (j + 1) * _CH

                @pl.when((j + 1) % 2 == 0)
                def _():
                    pltpu.async_copy(g_hbm.at[c, pl.ds(o, _CH)], r0buf, sem0)

                @pl.when((j + 1) % 2 == 1)
                def _():
                    pltpu.async_copy(g_hbm.at[c, pl.ds(o, _CH)], r1buf, sem1)

            o = base + j * _CH

            @pl.when(j % 2 == 0)
            def _():
                pltpu.make_async_copy(g_hbm.at[c, pl.ds(o, _CH)], r0buf,
                                      sem0).wait()
                pltpu.sync_copy(r0buf, acc.at[idx.at[j]], add=True)

            @pl.when(j % 2 == 1)
            def _():
                pltpu.make_async_copy(g_hbm.at[c, pl.ds(o, _CH)], r1buf,
                                      sem1).wait()
                pltpu.sync_copy(r1buf, acc.at[idx.at[j]], add=True)

            return carry

        lax.fori_loop(0, _SCH, body, 0)
        plsc.subcore_barrier()
        pltpu.sync_copy(acc.at[pl.ds(s * _NPS, _NPS)],
                        agg_hbm.at[c, pl.ds(s * _NPS, _NPS)])

    return k(gated_st, dst.reshape(16, _SCH, _CH), zeros_np)


def _sc_segmean(x_st, ids, zeros_g, zeros_c, ones_c):
    @functools.partial(
        pl.kernel, mesh=_sc_mesh(),
        out_type=[jax.ShapeDtypeStruct((2, NUM_GRAPHS, 128), F32),
                  jax.ShapeDtypeStruct((2, NUM_GRAPHS, 128), F32)],
        scratch_types=[pltpu.VMEM((_CH,), jnp.int32),
                       pltpu.VMEM((_CH, 128), F32),
                       pltpu.VMEM((_CH, 128), F32),
                       pltpu.VMEM_SHARED((NUM_GRAPHS, 128), F32),
                       pltpu.VMEM_SHARED((NUM_GRAPHS, 128), F32)],
    )
    def k(x_hbm, ids_hbm, zg_hbm, zc_hbm, ones_hbm, sum_hbm, cnt_hbm,
          idx_v, rows_v, ones_v, sacc, cacc):
        c = lax.axis_index("c")
        s = lax.axis_index("s")
        pltpu.sync_copy(zg_hbm.at[pl.ds(s * 32, 32)], sacc.at[pl.ds(s * 32, 32)])
        pltpu.sync_copy(zc_hbm.at[pl.ds(s * 32, 32)], cacc.at[pl.ds(s * 32, 32)])
        pltpu.sync_copy(ones_hbm, ones_v)
        plsc.subcore_barrier()

        def body(kk, carry):
            q = s + kk * 16

            @pl.when(q < _NCH)
            def _():
                r0 = q * _CH
                pltpu.sync_copy(ids_hbm.at[pl.ds(r0, _CH)], idx_v)
                pltpu.sync_copy(x_hbm.at[c, pl.ds(r0, _CH)], rows_v)
                pltpu.sync_copy(rows_v, sacc.at[idx_v], add=True)
                pltpu.sync_copy(ones_v, cacc.at[idx_v], add=True)

            return carry

        lax.fori_loop(0, (_NCH + 15) // 16, body, 0)
        plsc.subcore_barrier()
        pltpu.sync_copy(sacc.at[pl.ds(s * 32, 32)],
                        sum_hbm.at[c, pl.ds(s * 32, 32)])
        pltpu.sync_copy(cacc.at[pl.ds(s * 32, 32)],
                        cnt_hbm.at[c, pl.ds(s * 32, 32)])

    return k(x_st, ids, zeros_g, zeros_c, ones_c)


def _row(v):
    return v.reshape(1, -1)


def _ss_mat(sts, count, g, b):
    if not isinstance(sts, (list, tuple)):
        sts = [sts]
    n = count / len(sts)
    means, e2s = [], []
    for st in sts:
        m1 = st[0] / n
        means.append(st[2] + m1)
        e2s.append(st[1] / n - m1 * m1)
    mean = sum(means) / len(sts)
    var = sum(e2s[i] + (means[i] - mean) ** 2 for i in range(len(sts))) \
        / len(sts)
    scale = g * jax.lax.rsqrt(var + 1e-5)
    shift = b - mean * scale
    return jnp.concatenate([_row(scale), _row(shift),
                            jnp.zeros((6, NF), F32)], axis=0)


def _fold_conv(cp):
    K1 = cp["key_update"]["l1"]["w"]
    K1b = cp["key_update"]["l1"]["b"]
    M1 = cp["msg_update"]["l1"]["w"]
    M1b = cp["msg_update"]["l1"]["b"]
    Wq, bq = cp["q"]["w"], cp["q"]["b"]
    Wk, bk = cp["k"]["w"], cp["k"]["b"]
    Wv, bv = cp["v"]["w"], cp["v"]["b"]
    We, be = cp["e"]["w"], cp["e"]["b"]
    f = {}
    f["wdst"] = jnp.concatenate([Wq, Wk @ K1[:NF], Wv @ M1[:NF]], axis=1)
    f["bdst"] = _row(jnp.concatenate([bq, bk @ K1[:NF], bv @ M1[:NF]]))
    f["wsrc"] = jnp.concatenate([Wk @ K1[NF:2 * NF], Wv @ M1[NF:2 * NF]], axis=1)
    f["bsrc"] = _row(jnp.concatenate([bk @ K1[NF:2 * NF], bv @ M1[NF:2 * NF]]))
    f["ek"] = We @ K1[2 * NF:]
    f["em"] = We @ M1[2 * NF:]
    f["k2"] = cp["key_update"]["l2"]["w"]
    f["m2"] = cp["msg_update"]["l2"]["w"]
    f["bias"] = jnp.concatenate(
        [_row(be @ K1[2 * NF:] + K1b), _row(be @ M1[2 * NF:] + M1b),
         _row(cp["key_update"]["l2"]["b"]), _row(cp["msg_update"]["l2"]["b"]),
         jnp.zeros((4, NF), F32)], axis=0)
    f["wc"] = cp["concat"]["w"]
    f["bc"] = _row(cp["concat"]["b"])
    return f


def _fold_eu(ep):
    K1 = ep["key_update"]["l1"]["w"]
    K1b = ep["key_update"]["l1"]["b"]
    M1 = ep["msg_update"]["l1"]["w"]
    M1b = ep["msg_update"]["l1"]["b"]
    K1a, K1bb, K1c = K1[:NF], K1[NF:2 * NF], K1[2 * NF:]
    M1a, M1bb, M1c = M1[:NF], M1[NF:2 * NF], M1[2 * NF:]
    Wel = ep["edge_len"]["w"]
    bel = ep["edge_len"]["b"]
    WelN, WelL = Wel[:NF], Wel[NF:]
    lemb = ep["lemb"]
    f = {}
    f["wkc"] = ep["k"]["w"] @ K1a
    f["wvc"] = ep["v"]["w"] @ M1a
    f["wq"] = ep["q"]["w"]
    f["ak"] = ep["e"]["w"] @ K1c
    f["am"] = ep["e"]["w"] @ M1c
    f["k2"] = ep["key_update"]["l2"]["w"]
    f["m2"] = ep["msg_update"]["l2"]["w"]
    f["wc"] = ep["concat"]["w"]
    akb = ep["e"]["b"] @ K1c
    amb = ep["e"]["b"] @ M1c
    mk, mv, bk_rows, bm_rows = [], [], [], []
    for i in range(3):
        Wki, bki = ep["key_e%d" % i]["w"], ep["key_e%d" % i]["b"]
        Wvi, bvi = ep["value_e%d" % i]["w"], ep["value_e%d" % i]["b"]
        nlen_b = lemb[i] @ WelL + bel
        mk.append(WelN @ Wki @ K1bb)
        mv.append(WelN @ Wvi @ M1bb)
        bk_rows.append(_row(nlen_b @ Wki @ K1bb + bki @ K1bb
                            + ep["k"]["b"] @ K1a + akb + K1b))
        bm_rows.append(_row(nlen_b @ Wvi @ M1bb + bvi @ M1bb
                            + ep["v"]["b"] @ M1a + amb + M1b))
    zero5 = jnp.zeros((5, NF), F32)
    f["wcatk"] = [jnp.concatenate([f["wkc"], mk[i], f["ak"]], axis=0)
                  for i in range(3)]
    f["wcatv"] = [jnp.concatenate([f["wvc"], mv[i], f["am"]], axis=0)
                  for i in range(3)]
    f["biask"] = [jnp.concatenate(
        [bk_rows[i], zero5, _row(ep["q"]["b"]),
         _row(ep["key_update"]["l2"]["b"])], axis=0) for i in range(3)]
    f["biasv"] = [jnp.concatenate(
        [bm_rows[i], zero5, _row(ep["msg_update"]["l2"]["b"]),
         _row(3.0 * ep["concat"]["b"])], axis=0) for i in range(3)]
    return f


def _conv_layer(cp, f, node_st, ef, dst, src, zeros_np):
    tdp, tsp = _tc_prep(node_st, f["wdst"], f["bdst"], f["wsrc"], f["bsrc"])
    gd, gs = _sc_gather(tdp, tsp, dst, src)
    alpha, msg, st = _tc_conv_p1(gd, gs, ef, f["ek"], f["em"], f["k2"],
                                 f["m2"], f["bias"])
    ss = _ss_mat(st, float(N_EDGES), cp["bn_att"]["g"], cp["bn_att"]["b"])
    gated = _tc_conv_p2(alpha, msg, ss)
    agg = _sc_scatter(gated, dst, zeros_np)
    z, st2 = _tc_fina(agg, f["wc"], f["bc"])
    ss2 = _ss_mat(st2, float(N_NODES), cp["bn"]["g"], cp["bn"]["b"])
    return _tc_finb(node_st, z, ss2)


def kernel(x, edge_attr, edge_nei, kge_x, params, edge_index, batch, kge_batch):
    src = edge_index[0]
    dst = edge_index[1]
    xp = jnp.pad(x, ((0, 0), (0, 128 - x.shape[1])))
    wemb = jnp.pad(params["atom_embedding"]["w"], ((0, 128 - x.shape[1]), (0, 0)))
    nei9 = edge_nei.reshape(N_EDGES, 9)
    c_len = _row(jnp.linspace(-4.0, 0.0, BINS))
    c_ang = _row(jnp.linspace(-1.0, 1.0, BINS))
    zeros_np = jnp.zeros((_NPS, 128), F32)
    zeros_g = jnp.zeros((NUM_GRAPHS, 128), F32)
    ones_c = jnp.ones((_CH, 128), F32)

    ef = _tc_rbf_ef(edge_attr, c_len, params["rbf_lin"]["w"],
                    _row(params["rbf_lin"]["b"]))
    nl, na = _tc_rbf_nlna(nei9, edge_attr, c_len, c_ang,
                          params["rbf_lin"]["w"], _row(params["rbf_lin"]["b"]),
                          params["rbf_angle_lin"]["w"],
                          _row(params["rbf_angle_lin"]["b"]))
    node_st = _tc_embed(xp, wemb, _row(params["atom_embedding"]["b"]))

    with jax.default_matmul_precision("highest"):
        folds = [_fold_conv(params["att%d" % i]) for i in range(4)]
        fe = _fold_eu(params["edge_layer"])
    node_st = _conv_layer(params["att0"], folds[0], node_st, ef, dst, src,
                          zeros_np)

    ep = params["edge_layer"]
    alphas = []
    sts = []
    for i in range(3):
        a_i, st_i = _tc_eu_slot1(ef, nl, na, i, fe["wcatk"][i], fe["wq"],
                                 fe["k2"], fe["biask"][i])
        alphas.append(a_i)
        sts.append(st_i)
    ss = _ss_mat(sts, float(3 * N_EDGES), ep["bn_att"]["g"],
                 ep["bn_att"]["b"])
    s = None
    for i in range(3):
        s = _tc_eu_slot2(ef, nl, na, alphas[i], s, i, ss, fe["wcatv"][i],
                         fe["m2"], fe["biasv"][i])
    outp, st2 = _tc_eu_fin(s, fe["wc"], fe["biasv"][0])
    ss2 = _ss_mat(st2, float(N_EDGES), ep["bn"]["g"], ep["bn"]["b"])
    ef = _tc_eu_p3(ef, outp, ss2)

    for i in range(1, 4):
        node_st = _conv_layer(params["att%d" % i], folds[i], node_st, ef,
                              dst, src, zeros_np)

    gsum, gcnt = _sc_segmean(node_st, batch, zeros_g, zeros_g, ones_c)
    elem_st = _tc_proj(kge_x, params["project"]["w"],
                       _row(params["project"]["b"]))
    esum, ecnt = _sc_segmean(elem_st, kge_batch, zeros_g, zeros_g, ones_c)

    cg = params["cga"]
    ones_row = jnp.ones((1, NF), F32)
    hv = jnp.concatenate(
        [_row(cg["pa_w1"]), _row(cg["pa_w2"]), _row(cg["pa_b"]),
         _row(cg["ca2"]["b"]), cg["sa_w"][0, 0] * ones_row,
         cg["sa_w"][1, 0] * ones_row, cg["sa_b"][0] * ones_row,
         _row(cg["conv"]["b"]), _row(params["fc"]["b"]),
         _row(params["fc_out"]["w"][:, 0]),
         params["fc_out"]["b"][0] * ones_row,
         jnp.zeros((5, NF), F32)], axis=0)
    out = _tc_head(gsum, gcnt, esum, ecnt, cg["ca1"]["w"],
                   _row(cg["ca1"]["b"]), cg["ca2"]["w"], hv,
                   cg["conv"]["w"], params["fc"]["w"])
    return jnp.squeeze(out)
